# baseline (speedup 1.0000x reference)
import jax
import jax.numpy as jnp
from jax.experimental import pallas as pl
from jax.experimental.pallas import tpu as pltpu

_BN = 32768
_KC = 256
_SUB = 8


def _loss_kernel(et_ref, c_ref, out_ref, caug_ref):
    i = pl.program_id(0)

    @pl.when(i == 0)
    def _build():
        c = c_ref[...]
        c_sq = jnp.sum(c * c, axis=1, keepdims=True)
        hi = c_sq.astype(jnp.bfloat16)
        lo = (c_sq - hi.astype(jnp.float32)).astype(jnp.bfloat16)
        caug_ref[...] = jnp.concatenate(
            [(-2.0 * c).astype(jnp.bfloat16), hi, lo], axis=1)

    et = et_ref[...]
    bn = et.shape[1]
    et_aug = jnp.concatenate(
        [et.astype(jnp.bfloat16),
         jnp.ones((2, bn), jnp.bfloat16)], axis=0)
    c_aug = caug_ref[...]
    k = c_aug.shape[0]

    m_acc = None
    for j in range(k // _KC):
        cj = c_aug[j * _KC:(j + 1) * _KC, :]
        pj = jax.lax.dot_general(
            cj, et_aug, (((1,), (0,)), ((), ())),
            preferred_element_type=jnp.float32)
        tiles = [pj[t * _SUB:(t + 1) * _SUB, :] for t in range(_KC // _SUB)]
        while len(tiles) > 1:
            tiles = [jnp.minimum(tiles[t], tiles[t + 1])
                     for t in range(0, len(tiles) - 1, 2)] + (
                         [tiles[-1]] if len(tiles) % 2 else [])
        mj = tiles[0]
        m_acc = mj if m_acc is None else jnp.minimum(m_acc, mj)
    m_col = jnp.min(m_acc, axis=0, keepdims=True)
    e_sq = jnp.sum(et * et, axis=0, keepdims=True)
    partial = jnp.sum(m_col + e_sq).reshape(1, 1)

    @pl.when(i == 0)
    def _init():
        out_ref[...] = jnp.zeros_like(out_ref)

    out_ref[...] += partial


def kernel(embedded, centers):
    n, d = embedded.shape
    k = centers.shape[0]
    et = embedded.T
    grid = n // _BN
    total = pl.pallas_call(
        _loss_kernel,
        grid=(grid,),
        in_specs=[
            pl.BlockSpec((d, _BN), lambda i: (0, i)),
            pl.BlockSpec((k, d), lambda i: (0, 0)),
        ],
        out_specs=pl.BlockSpec((1, 1), lambda i: (0, 0)),
        out_shape=jax.ShapeDtypeStruct((1, 1), jnp.float32),
        scratch_shapes=[pltpu.VMEM((k, d + 2), jnp.bfloat16)],
    )(et, centers)
    return total[0, 0] / n
